# Initial kernel scaffold; baseline (speedup 1.0000x reference)
import jax
import jax.numpy as jnp
from jax.experimental import pallas as pl


def kernel(xyz, feature):
    raise NotImplementedError("write your pallas kernel here")


# speedup vs baseline: 37.1421x; 37.1421x over previous
import functools

import jax
import jax.numpy as jnp
from jax import lax
from jax.experimental import pallas as pl
from jax.experimental.pallas import tpu as pltpu
from jax.experimental.pallas import tpu_sc as plsc

_B, _N, _C, _K = 16, 8192, 128, 512
_BN = 512


def _fps_body(x_ref, y_ref, z_ref, nx_ref, ny_ref, nz_ref, gidx_ref, dist_ref):
    nb = _N // _BN
    dist_ref[...] = jnp.full((_B, _N), 1e10, jnp.float32)
    lane = lax.broadcasted_iota(jnp.int32, (_B, _BN), 1)
    klane = lax.broadcasted_iota(jnp.int32, (_B, _K), 1)

    def step(k, carry):
        far, cx, cy, cz, nx, ny, nz, gx = carry
        sel = klane == k
        nx = jnp.where(sel, cx, nx)
        ny = jnp.where(sel, cy, ny)
        nz = jnp.where(sel, cz, nz)
        gx = jnp.where(sel, far, gx)
        bv = jnp.full((_B, _BN), -1.0, jnp.float32)
        bi = jnp.zeros((_B, _BN), jnp.int32)
        bx = jnp.zeros((_B, _BN), jnp.float32)
        by = jnp.zeros((_B, _BN), jnp.float32)
        bz = jnp.zeros((_B, _BN), jnp.float32)
        for j in range(nb):
            sl = pl.ds(j * _BN, _BN)
            xj = x_ref[:, sl]
            yj = y_ref[:, sl]
            zj = z_ref[:, sl]
            dx = xj - cx
            dy = yj - cy
            dz = zj - cz
            d = dx * dx + dy * dy + dz * dz
            nd = jnp.minimum(dist_ref[:, sl], d)
            dist_ref[:, sl] = nd
            m = nd > bv
            bv = jnp.where(m, nd, bv)
            bi = jnp.where(m, lane + (j * _BN), bi)
            bx = jnp.where(m, xj, bx)
            by = jnp.where(m, yj, by)
            bz = jnp.where(m, zj, bz)
        mv = jnp.max(bv, axis=1, keepdims=True)
        cand = jnp.where(bv == mv, bi, _N)
        fi = jnp.min(cand, axis=1, keepdims=True)
        m3 = bi == fi
        ncx = jnp.sum(jnp.where(m3, bx, 0.0), axis=1, keepdims=True)
        ncy = jnp.sum(jnp.where(m3, by, 0.0), axis=1, keepdims=True)
        ncz = jnp.sum(jnp.where(m3, bz, 0.0), axis=1, keepdims=True)
        return fi, ncx, ncy, ncz, nx, ny, nz, gx

    far0 = jnp.zeros((_B, 1), jnp.int32)
    z0i = klane + lax.broadcasted_iota(jnp.int32, (_B, _K), 0)
    z0f = z0i.astype(jnp.float32)
    init = (far0, x_ref[:, 0:1], y_ref[:, 0:1], z_ref[:, 0:1], z0f, z0f, z0f, z0i)
    _, _, _, _, nx, ny, nz, gx = lax.fori_loop(0, _K, step, init)
    nx_ref[...] = nx
    ny_ref[...] = ny
    nz_ref[...] = nz
    gidx_ref[...] = gx + lax.broadcasted_iota(jnp.int32, (_B, _K), 0) * _N


_fps = pl.pallas_call(
    _fps_body,
    out_shape=[
        jax.ShapeDtypeStruct((_B, _K), jnp.float32),
        jax.ShapeDtypeStruct((_B, _K), jnp.float32),
        jax.ShapeDtypeStruct((_B, _K), jnp.float32),
        jax.ShapeDtypeStruct((_B, _K), jnp.int32),
    ],
    scratch_shapes=[pltpu.VMEM((_B, _N), jnp.float32)],
)


def _make_gather():
    info = plsc.get_sparse_core_info()
    nc, ns = info.num_cores, info.num_subcores
    nw = nc * ns
    rows = _B * _K
    rpw = rows // nw
    chunks = rpw // 128
    mesh = plsc.VectorSubcoreMesh(core_axis_name="c", subcore_axis_name="s")

    @functools.partial(
        pl.kernel,
        mesh=mesh,
        out_type=jax.ShapeDtypeStruct((rows, _C), jnp.float32),
        scratch_types=[
            pltpu.VMEM((chunks, 128), jnp.int32),
            pltpu.VMEM((rpw, _C), jnp.float32),
            pltpu.SemaphoreType.DMA,
        ],
    )
    def gather(table_hbm, idx_hbm, out_hbm, idx_v, rows_v, sem):
        wid = lax.axis_index("s") * nc + lax.axis_index("c")
        base = wid * rpw
        pltpu.sync_copy(idx_hbm.at[pl.ds(wid * chunks, chunks)], idx_v)
        copies = [
            pltpu.async_copy(
                table_hbm.at[idx_v.at[t]],
                rows_v.at[pl.ds(t * 128, 128)],
                sem,
            )
            for t in range(chunks)
        ]
        for c in copies:
            c.wait()
        pltpu.sync_copy(rows_v, out_hbm.at[pl.ds(base, rpw)])

    return gather


def kernel(xyz, feature):
    x = xyz[:, 0, :]
    y = xyz[:, 1, :]
    z = xyz[:, 2, :]
    nx, ny, nz, gidx = _fps(x, y, z)
    node_static = jnp.stack([nx, ny, nz], axis=1)
    table = feature.transpose(0, 2, 1).reshape(_B * _N, _C)
    idx2d = gidx.reshape(-1, 128)
    rows = _make_gather()(table, idx2d)
    node_feature = rows.reshape(_B, _K, _C).transpose(0, 2, 1)
    return node_static, node_feature
